# baseline (speedup 1.0000x reference)
import functools

import jax
import jax.numpy as jnp
import numpy as np
from jax import lax
from jax.experimental import pallas as pl
from jax.experimental.pallas import tpu as pltpu
from jax.experimental.pallas import tpu_sc as plsc

_I32_MIN = np.int32(-(2**31))
_I32_MAXMAG = np.int32(0x7FFFFFFF)

_NBINS = 16384
_SC_CORES = 2
_SC_SUBCORES = 16
_NQ = 4
_CR = 64


def _sc_hist_kernel(keys_hbm, hist_hbm, buf0, buf1, hist_v, sem0, sem1,
                    *, H, QROWS):
    s = lax.axis_index("s")
    c = lax.axis_index("c")
    img = s // 2
    q = (s % 2) * 2 + c
    r0 = q * QROWS

    def _zero(i, z):
        hist_v[pl.ds(i * 16, 16)] = jnp.zeros((16,), jnp.int32)
        return z

    lax.fori_loop(0, _NBINS // 16, _zero, 0)

    ones = jnp.ones((16,), jnp.int32)
    bufs = (buf0, buf1)
    sems = (sem0, sem1)
    nch = QROWS // _CR

    copies = [None] * nch
    copies[0] = pltpu.async_copy(
        keys_hbm.at[img, pl.ds(r0, _CR)], buf0, sem0)

    zacc = jnp.zeros((16,), jnp.int32)
    for k in range(nch):
        if k + 1 < nch:
            copies[k + 1] = pltpu.async_copy(
                keys_hbm.at[img, pl.ds(r0 + (k + 1) * _CR, _CR)],
                bufs[(k + 1) % 2], sems[(k + 1) % 2])
        copies[k].wait()
        buf = bufs[k % 2]

        def _row(r, z):
            for j in range(H // 16):
                v = buf[r, pl.ds(j * 16, 16)]
                nz = v != 0
                bins = (v >> 18) + 8192
                plsc.addupdate_scatter(hist_v, [bins], ones, mask=nz)
                z = z + jnp.where(nz, 0, 1)
            return z

        zacc = lax.fori_loop(0, _CR, _row, zacc)

    nzeros = jnp.sum(zacc)
    lane0 = lax.broadcasted_iota(jnp.int32, (16,), 0) == 0
    zslot = hist_v[pl.ds(8192, 16)]
    hist_v[pl.ds(8192, 16)] = zslot + jnp.where(lane0, nzeros, 0)

    pltpu.sync_copy(hist_v, hist_hbm.at[q, img])


def _keys_kernel(boxes_ref, x_ref, key_ref, *, W, H, NB):
    i = pl.program_id(0)
    x = x_ref[0]

    rows = lax.broadcasted_iota(jnp.int32, (W, 1), 0)
    cols = lax.broadcasted_iota(jnp.int32, (1, H), 1)
    mask = jnp.zeros((W, H), dtype=jnp.bool_)
    for j in range(NB):
        x1 = boxes_ref[i, j, 0] // 8
        y1 = boxes_ref[i, j, 1] // 8
        x2 = boxes_ref[i, j, 2] // 8
        y2 = boxes_ref[i, j, 3] // 8
        rin = (rows >= y1) & (rows < y2)
        cin = (cols >= x1) & (cols < x2)
        mask = mask | (rin & cin)

    lab = jnp.where(mask, jnp.float32(0.0), x)
    bits = lax.bitcast_convert_type(lab, jnp.int32)
    key = jnp.where(bits >= 0, bits, bits ^ _I32_MAXMAG)
    key_ref[0] = jnp.where(bits == _I32_MIN, jnp.int32(0), key)


def _select_kernel(keys_a_ref, keys_b_ref, boxes_ref, hist_a_ref, hist_b_ref,
                   kstar_ref, idxstar_ref, arr_a_scr, arr_b_scr,
                   *, B, W, H, NB):
    HB = B // 2
    keys_a = keys_a_ref[...]
    keys_b = keys_b_ref[...]

    b = boxes_ref[...] // 8
    wdt = b[:, :, 2] - b[:, :, 0]
    hgt = b[:, :, 3] - b[:, :, 1]
    K = jnp.sum(wdt * hgt, axis=1, keepdims=True)[..., None]

    h = jnp.concatenate(
        [jnp.sum(hist_a_ref[...], axis=0), jnp.sum(hist_b_ref[...], axis=0)],
        axis=0)
    K2 = K[:, :, 0]
    bins_iota = lax.broadcasted_iota(jnp.int32, (1, _NBINS), 1)

    def _bucket_count_ge(cand):
        return jnp.sum(jnp.where(bins_iota >= cand, h, 0), axis=1,
                       keepdims=True)

    def _bucket_body(it, p):
        bit = lax.shift_left(jnp.int32(1), 13 - it)
        cand = p | bit
        return jnp.where(_bucket_count_ge(cand) >= K2, cand, p)

    bstar = lax.fori_loop(0, 14, _bucket_body, jnp.zeros((B, 1), jnp.int32))
    T0 = lax.shift_left(bstar - 8192, 18)[..., None]

    def _split_count(pred_fn, cand):
        ca = jnp.sum(pred_fn(keys_a, cand[:HB]).astype(jnp.int32),
                     axis=(1, 2), keepdims=True)
        cb = jnp.sum(pred_fn(keys_b, cand[HB:]).astype(jnp.int32),
                     axis=(1, 2), keepdims=True)
        return jnp.concatenate([ca, cb], axis=0)

    def _key_body(it, T):
        bit = lax.shift_left(jnp.int32(1), 17 - it)
        cand = T | bit
        return jnp.where(_split_count(lambda k, c: k >= c, cand) >= K,
                         cand, T)

    kstar = lax.fori_loop(0, 18, _key_body, T0)

    c_gt = _split_count(lambda k, c: k > c, kstar)
    c_ge = _split_count(lambda k, c: k >= c, kstar)
    t = K - c_gt

    kstar_ref[...] = kstar

    straddle = jnp.sum((c_ge > K).astype(jnp.int32)) > 0

    @pl.when(jnp.logical_not(straddle))
    def _():
        idxstar_ref[...] = jnp.full((B, 1, 1), W * H, jnp.int32)

    @pl.when(straddle)
    def _():
        rows = lax.broadcasted_iota(jnp.int32, (1, W, H), 1)
        cols = lax.broadcasted_iota(jnp.int32, (1, W, H), 2)
        idx = rows * H + cols
        arr_a_scr[...] = jnp.where(keys_a == kstar[:HB], idx, _I32_MAXMAG)
        arr_b_scr[...] = jnp.where(keys_b == kstar[HB:], idx, _I32_MAXMAG)

        def _idx_body(it, p):
            bit = lax.shift_left(jnp.int32(1), 17 - it)
            cand = p | bit
            ca = jnp.sum((arr_a_scr[...] < cand[:HB]).astype(jnp.int32),
                         axis=(1, 2), keepdims=True)
            cb = jnp.sum((arr_b_scr[...] < cand[HB:]).astype(jnp.int32),
                         axis=(1, 2), keepdims=True)
            c = jnp.concatenate([ca, cb], axis=0)
            return jnp.where(c < t, cand, p)

        idxstar_ref[...] = lax.fori_loop(0, 18, _idx_body,
                                         jnp.zeros((B, 1, 1), jnp.int32))


def _final_kernel(kstar_ref, idxstar_ref, x_ref, keys_a_ref, keys_b_ref,
                  loss_ref, label_ref, *, W, H, HB):
    i = pl.program_id(0)
    x = x_ref[0]
    key = jnp.where(i < HB, keys_a_ref[0], keys_b_ref[0])
    ks = kstar_ref[i]
    istar = idxstar_ref[i]

    rows = lax.broadcasted_iota(jnp.int32, (W, 1), 0)
    cols = lax.broadcasted_iota(jnp.int32, (1, H), 1)
    idx = rows * H + cols

    sel = (key > ks) | ((key == ks) & (idx <= istar))
    lab = lax.bitcast_convert_type(
        jnp.where(key >= 0, key, key ^ _I32_MAXMAG), jnp.float32)
    label = jnp.where(sel, jnp.float32(1.0), lab)
    label_ref[0] = label

    @pl.when(i == 0)
    def _():
        loss_ref[...] = jnp.zeros((1, 1), jnp.float32)

    loss_ref[...] += jnp.sum(jnp.abs(x - label), keepdims=True)


def _make_keys(x_half, boxes_half, W, H, NB, HB):
    return pl.pallas_call(
        functools.partial(_keys_kernel, W=W, H=H, NB=NB),
        grid_spec=pltpu.PrefetchScalarGridSpec(
            num_scalar_prefetch=1,
            grid=(HB,),
            in_specs=[pl.BlockSpec((1, W, H), lambda i, b: (i, 0, 0))],
            out_specs=pl.BlockSpec((1, W, H), lambda i, b: (i, 0, 0)),
        ),
        out_shape=jax.ShapeDtypeStruct((HB, W, H), jnp.int32),
    )(boxes_half, x_half)


def _make_hist(keys_half, W, H, HB):
    return pl.kernel(
        functools.partial(_sc_hist_kernel, H=H, QROWS=W // _NQ),
        out_type=jax.ShapeDtypeStruct((_NQ, HB, _NBINS), jnp.int32),
        mesh=plsc.VectorSubcoreMesh(
            core_axis_name="c", subcore_axis_name="s",
            num_cores=_SC_CORES, num_subcores=_SC_SUBCORES),
        scratch_types=[
            pltpu.VMEM((_CR, H), jnp.int32),
            pltpu.VMEM((_CR, H), jnp.int32),
            pltpu.VMEM((_NBINS,), jnp.int32),
            pltpu.SemaphoreType.DMA,
            pltpu.SemaphoreType.DMA,
        ],
        compiler_params=pltpu.CompilerParams(needs_layout_passes=False),
    )(keys_half)


def kernel(x, boxes):
    B, W, H = x.shape
    NB = boxes.shape[1]
    HB = B // 2

    keys_a = _make_keys(x[:HB], boxes[:HB], W, H, NB, HB)
    hist_a = _make_hist(keys_a, W, H, HB)
    keys_b = _make_keys(x[HB:], boxes[HB:], W, H, NB, HB)
    hist_b = _make_hist(keys_b, W, H, HB)

    kstar, idxstar = pl.pallas_call(
        functools.partial(_select_kernel, B=B, W=W, H=H, NB=NB),
        grid=(1,),
        in_specs=[
            pl.BlockSpec((HB, W, H), lambda i: (0, 0, 0)),
            pl.BlockSpec((HB, W, H), lambda i: (0, 0, 0)),
            pl.BlockSpec((B, NB, 4), lambda i: (0, 0, 0)),
            pl.BlockSpec((_NQ, HB, _NBINS), lambda i: (0, 0, 0)),
            pl.BlockSpec((_NQ, HB, _NBINS), lambda i: (0, 0, 0)),
        ],
        out_specs=[
            pl.BlockSpec((B, 1, 1), lambda i: (0, 0, 0)),
            pl.BlockSpec((B, 1, 1), lambda i: (0, 0, 0)),
        ],
        scratch_shapes=[
            pltpu.VMEM((HB, W, H), jnp.int32),
            pltpu.VMEM((HB, W, H), jnp.int32),
        ],
        out_shape=[
            jax.ShapeDtypeStruct((B, 1, 1), jnp.int32),
            jax.ShapeDtypeStruct((B, 1, 1), jnp.int32),
        ],
    )(keys_a, keys_b, boxes, hist_a, hist_b)

    loss_sum, label = pl.pallas_call(
        functools.partial(_final_kernel, W=W, H=H, HB=HB),
        grid_spec=pltpu.PrefetchScalarGridSpec(
            num_scalar_prefetch=2,
            grid=(B,),
            in_specs=[
                pl.BlockSpec((1, W, H), lambda i, a, c: (i, 0, 0)),
                pl.BlockSpec((1, W, H),
                             lambda i, a, c: (jnp.minimum(i, HB - 1), 0, 0)),
                pl.BlockSpec((1, W, H),
                             lambda i, a, c: (jnp.maximum(i - HB, 0), 0, 0)),
            ],
            out_specs=[
                pl.BlockSpec((1, 1), lambda i, a, c: (0, 0)),
                pl.BlockSpec((1, W, H), lambda i, a, c: (i, 0, 0)),
            ],
        ),
        out_shape=[
            jax.ShapeDtypeStruct((1, 1), jnp.float32),
            jax.ShapeDtypeStruct((B, W, H), jnp.float32),
        ],
    )(kstar.reshape(B), idxstar.reshape(B), x, keys_a, keys_b)

    loss = loss_sum[0, 0] / jnp.float32(B * W * H)
    return (loss, x, label)


# speedup vs baseline: 1.0287x; 1.0287x over previous
import functools

import jax
import jax.numpy as jnp
import numpy as np
from jax import lax
from jax.experimental import pallas as pl
from jax.experimental.pallas import tpu as pltpu
from jax.experimental.pallas import tpu_sc as plsc

_I32_MIN = np.int32(-(2**31))
_I32_MAXMAG = np.int32(0x7FFFFFFF)

_NBINS = 16384
_SC_CORES = 2
_SC_SUBCORES = 16
_CR = 64


def _sc_hist_kernel(keys_hbm, hist_hbm, buf0, buf1, hist_v, sem0, sem1,
                    *, H, HALF_ROWS):
    img = lax.axis_index("s")
    half = lax.axis_index("c")
    r0 = half * HALF_ROWS

    def _zero(i, c):
        hist_v[pl.ds(i * 16, 16)] = jnp.zeros((16,), jnp.int32)
        return c

    lax.fori_loop(0, _NBINS // 16, _zero, 0)

    ones = jnp.ones((16,), jnp.int32)
    bufs = (buf0, buf1)
    sems = (sem0, sem1)
    nch = HALF_ROWS // _CR

    copies = [None] * nch
    copies[0] = pltpu.async_copy(
        keys_hbm.at[img, pl.ds(r0, _CR)], buf0, sem0)

    zacc = jnp.zeros((16,), jnp.int32)
    for k in range(nch):
        if k + 1 < nch:
            copies[k + 1] = pltpu.async_copy(
                keys_hbm.at[img, pl.ds(r0 + (k + 1) * _CR, _CR)],
                bufs[(k + 1) % 2], sems[(k + 1) % 2])
        copies[k].wait()
        buf = bufs[k % 2]

        def _row(r, z):
            for j in range(H // 16):
                v = buf[r, pl.ds(j * 16, 16)]
                nz = v != 0
                bins = (v >> 18) + 8192
                plsc.addupdate_scatter(hist_v, [bins], ones, mask=nz)
                z = z + jnp.where(nz, 0, 1)
            return z

        zacc = lax.fori_loop(0, _CR, _row, zacc)

    nzeros = jnp.sum(zacc)
    lane0 = lax.broadcasted_iota(jnp.int32, (16,), 0) == 0
    zslot = hist_v[pl.ds(8192, 16)]
    hist_v[pl.ds(8192, 16)] = zslot + jnp.where(lane0, nzeros, 0)

    pltpu.sync_copy(hist_v, hist_hbm.at[half, img])


def _keys_kernel(boxes_ref, x_ref, key_ref, *, W, H, NB):
    i = pl.program_id(0)
    x = x_ref[0]

    rows = lax.broadcasted_iota(jnp.int32, (W, 1), 0)
    cols = lax.broadcasted_iota(jnp.int32, (1, H), 1)
    mask = jnp.zeros((W, H), dtype=jnp.bool_)
    for j in range(NB):
        x1 = boxes_ref[i, j, 0] // 8
        y1 = boxes_ref[i, j, 1] // 8
        x2 = boxes_ref[i, j, 2] // 8
        y2 = boxes_ref[i, j, 3] // 8
        rin = (rows >= y1) & (rows < y2)
        cin = (cols >= x1) & (cols < x2)
        mask = mask | (rin & cin)

    lab = jnp.where(mask, jnp.float32(0.0), x)
    bits = lax.bitcast_convert_type(lab, jnp.int32)
    key = jnp.where(bits >= 0, bits, bits ^ _I32_MAXMAG)
    key_ref[0] = jnp.where(bits == _I32_MIN, jnp.int32(0), key)


def _select_kernel(keys_ref, boxes_ref, hist_ref, kstar_ref, idxstar_ref,
                   arr_scr, *, B, W, H, NB):
    keys = keys_ref[...]

    b = boxes_ref[...] // 8
    wdt = b[:, :, 2] - b[:, :, 0]
    hgt = b[:, :, 3] - b[:, :, 1]
    K = jnp.sum(wdt * hgt, axis=1, keepdims=True)[..., None]

    h = hist_ref[0] + hist_ref[1]
    K2 = K[:, :, 0]
    bins_iota = lax.broadcasted_iota(jnp.int32, (1, _NBINS), 1)

    def _bucket_count_ge(cand):
        return jnp.sum(jnp.where(bins_iota >= cand, h, 0), axis=1,
                       keepdims=True)

    def _bucket_body(it, p):
        bit = lax.shift_left(jnp.int32(1), 13 - it)
        cand = p | bit
        return jnp.where(_bucket_count_ge(cand) >= K2, cand, p)

    bstar = lax.fori_loop(0, 14, _bucket_body, jnp.zeros((B, 1), jnp.int32))
    T0 = lax.shift_left(bstar - 8192, 18)[..., None]

    def _count_ge(cand):
        return jnp.sum((keys >= cand).astype(jnp.int32), axis=(1, 2),
                       keepdims=True)

    def _key_body(it, T):
        bit = lax.shift_left(jnp.int32(1), 17 - it)
        cand = T | bit
        return jnp.where(_count_ge(cand) >= K, cand, T)

    kstar = lax.fori_loop(0, 18, _key_body, T0)

    c_gt = jnp.sum((keys > kstar).astype(jnp.int32), axis=(1, 2),
                   keepdims=True)
    c_ge = jnp.sum((keys >= kstar).astype(jnp.int32), axis=(1, 2),
                   keepdims=True)
    t = K - c_gt

    kstar_ref[...] = kstar

    straddle = jnp.sum((c_ge > K).astype(jnp.int32)) > 0

    @pl.when(jnp.logical_not(straddle))
    def _():
        idxstar_ref[...] = jnp.full((B, 1, 1), W * H, jnp.int32)

    @pl.when(straddle)
    def _():
        rows = lax.broadcasted_iota(jnp.int32, (1, W, H), 1)
        cols = lax.broadcasted_iota(jnp.int32, (1, W, H), 2)
        idx = rows * H + cols
        arr_scr[...] = jnp.where(keys == kstar, idx, _I32_MAXMAG)

        def _idx_body(it, p):
            bit = lax.shift_left(jnp.int32(1), 17 - it)
            cand = p | bit
            c = jnp.sum((arr_scr[...] < cand).astype(jnp.int32), axis=(1, 2),
                        keepdims=True)
            return jnp.where(c < t, cand, p)

        idxstar_ref[...] = lax.fori_loop(0, 18, _idx_body,
                                         jnp.zeros((B, 1, 1), jnp.int32))


def _final_kernel(kstar_ref, idxstar_ref, x_ref, key_ref, loss_ref, label_ref,
                  *, W, H):
    i = pl.program_id(0)
    x = x_ref[0]
    key = key_ref[0]
    ks = kstar_ref[i]
    istar = idxstar_ref[i]

    rows = lax.broadcasted_iota(jnp.int32, (W, 1), 0)
    cols = lax.broadcasted_iota(jnp.int32, (1, H), 1)
    idx = rows * H + cols

    sel = (key > ks) | ((key == ks) & (idx <= istar))
    lab = lax.bitcast_convert_type(
        jnp.where(key >= 0, key, key ^ _I32_MAXMAG), jnp.float32)
    label = jnp.where(sel, jnp.float32(1.0), lab)
    label_ref[0] = label

    @pl.when(i == 0)
    def _():
        loss_ref[...] = jnp.zeros((1, 1), jnp.float32)

    loss_ref[...] += jnp.sum(jnp.abs(x - label), keepdims=True)


def kernel(x, boxes):
    B, W, H = x.shape
    NB = boxes.shape[1]
    N = W * H

    keys = pl.pallas_call(
        functools.partial(_keys_kernel, W=W, H=H, NB=NB),
        grid_spec=pltpu.PrefetchScalarGridSpec(
            num_scalar_prefetch=1,
            grid=(B,),
            in_specs=[pl.BlockSpec((1, W, H), lambda i, b: (i, 0, 0))],
            out_specs=pl.BlockSpec((1, W, H), lambda i, b: (i, 0, 0)),
        ),
        out_shape=jax.ShapeDtypeStruct((B, W, H), jnp.int32),
    )(boxes, x)

    hist = pl.kernel(
        functools.partial(_sc_hist_kernel, H=H, HALF_ROWS=W // _SC_CORES),
        out_type=jax.ShapeDtypeStruct((_SC_CORES, B, _NBINS), jnp.int32),
        mesh=plsc.VectorSubcoreMesh(
            core_axis_name="c", subcore_axis_name="s",
            num_cores=_SC_CORES, num_subcores=_SC_SUBCORES),
        scratch_types=[
            pltpu.VMEM((_CR, H), jnp.int32),
            pltpu.VMEM((_CR, H), jnp.int32),
            pltpu.VMEM((_NBINS,), jnp.int32),
            pltpu.SemaphoreType.DMA,
            pltpu.SemaphoreType.DMA,
        ],
        compiler_params=pltpu.CompilerParams(needs_layout_passes=False),
    )(keys)

    kstar, idxstar = pl.pallas_call(
        functools.partial(_select_kernel, B=B, W=W, H=H, NB=NB),
        grid=(1,),
        in_specs=[
            pl.BlockSpec((B, W, H), lambda i: (0, 0, 0)),
            pl.BlockSpec((B, NB, 4), lambda i: (0, 0, 0)),
            pl.BlockSpec((_SC_CORES, B, _NBINS), lambda i: (0, 0, 0)),
        ],
        out_specs=[
            pl.BlockSpec((B, 1, 1), lambda i: (0, 0, 0)),
            pl.BlockSpec((B, 1, 1), lambda i: (0, 0, 0)),
        ],
        scratch_shapes=[pltpu.VMEM((B, W, H), jnp.int32)],
        out_shape=[
            jax.ShapeDtypeStruct((B, 1, 1), jnp.int32),
            jax.ShapeDtypeStruct((B, 1, 1), jnp.int32),
        ],
    )(keys, boxes, hist)

    loss_sum, label = pl.pallas_call(
        functools.partial(_final_kernel, W=W, H=H),
        grid_spec=pltpu.PrefetchScalarGridSpec(
            num_scalar_prefetch=2,
            grid=(B,),
            in_specs=[
                pl.BlockSpec((1, W, H), lambda i, a, c: (i, 0, 0)),
                pl.BlockSpec((1, W, H), lambda i, a, c: (i, 0, 0)),
            ],
            out_specs=[
                pl.BlockSpec((1, 1), lambda i, a, c: (0, 0)),
                pl.BlockSpec((1, W, H), lambda i, a, c: (i, 0, 0)),
            ],
        ),
        out_shape=[
            jax.ShapeDtypeStruct((1, 1), jnp.float32),
            jax.ShapeDtypeStruct((B, W, H), jnp.float32),
        ],
    )(kstar.reshape(B), idxstar.reshape(B), x, keys)

    loss = loss_sum[0, 0] / jnp.float32(B * W * H)
    return (loss, x, label)
